# baseline (speedup 1.0000x reference)
import jax
import jax.numpy as jnp
from jax import lax
from jax.experimental import pallas as pl
from jax.experimental.pallas import tpu as pltpu

_B, _H, _R, _C, _K, _S = 2, 8, 16, 128, 64, 512
_N = _B * _R * _C
_HG = 2


def _vq_body(vecs_ref, csum_ref, ccnt_ref, vq_ref, z_ref, e_ref, commit_ref,
             iota_s):
    g = pl.program_id(0)

    @pl.when(g == 0)
    def _():
        iota_s[...] = lax.broadcasted_iota(jnp.int32, (_S, _N), 0).astype(
            jnp.float32)

    iota0 = iota_s[...]
    esum = None
    for hh in range(_HG):
        c = csum_ref[hh] / jnp.maximum(ccnt_ref[hh], 0.01)
        c2 = -2.0 * c
        cn = jnp.sum(c * c, axis=1, keepdims=True)
        v = jnp.concatenate(
            [vecs_ref[b, hh, r] for b in range(_B) for r in range(_R)],
            axis=1)
        dot2 = lax.dot_general(c2, v, (((1,), (0,)), ((), ())),
                               preferred_element_type=jnp.float32)
        vnorm = jnp.sum(v * v, axis=0, keepdims=True)
        d2 = (vnorm + dot2) + cn
        mind = jnp.min(d2, axis=0, keepdims=True)
        zf = jnp.min(jnp.where(d2 == mind, iota0, jnp.float32(_S)),
                     axis=0, keepdims=True)
        onehot = (iota0 == zf).astype(jnp.float32)
        cz = lax.dot_general(c, onehot, (((0,), (0,)), ((), ())),
                             preferred_element_type=jnp.float32)
        for b in range(_B):
            for r in range(_R):
                n0 = (b * _R + r) * _C
                vq_ref[b, hh, r] = cz[:, n0:n0 + _C]
        z_ref[:, hh] = zf.astype(jnp.int32).reshape(_B, _R, _C)
        e = jnp.maximum(mind, 0.0)
        e_ref[:, hh] = e.reshape(_B, _R, _C)
        s = jnp.sum(e)
        esum = s if esum is None else esum + s
    prev = jnp.where(g == 0, 0.0, commit_ref[0, 0])
    commit_ref[0, 0] = prev + esum


def kernel(vecs, c_sum, c_count):
    vt = jnp.swapaxes(vecs, 3, 4)
    ccnt = c_count.reshape(_H, _S, 1)

    def im_v(g):
        return (0, g, 0, 0, 0)

    def im_cb(g):
        return (g, 0, 0)

    def im_ze(g):
        return (0, g, 0, 0)

    vq, z_out, e_out, commit = pl.pallas_call(
        _vq_body,
        grid=(_H // _HG,),
        in_specs=[
            pl.BlockSpec((_B, _HG, _R, _K, _C), im_v),
            pl.BlockSpec((_HG, _S, _K), im_cb),
            pl.BlockSpec((_HG, _S, 1), im_cb),
        ],
        out_specs=[
            pl.BlockSpec((_B, _HG, _R, _K, _C), im_v),
            pl.BlockSpec((_B, _HG, _R, _C), im_ze),
            pl.BlockSpec((_B, _HG, _R, _C), im_ze),
            pl.BlockSpec((1, 1), lambda g: (0, 0), memory_space=pltpu.SMEM),
        ],
        out_shape=[
            jax.ShapeDtypeStruct((_B, _H, _R, _K, _C), jnp.float32),
            jax.ShapeDtypeStruct((_B, _H, _R, _C), jnp.int32),
            jax.ShapeDtypeStruct((_B, _H, _R, _C), jnp.float32),
            jax.ShapeDtypeStruct((1, 1), jnp.float32),
        ],
        scratch_shapes=[
            pltpu.VMEM((_S, _N), jnp.float32),
        ],
    )(vt, c_sum, ccnt)

    vecs_hat = jnp.swapaxes(vq, 3, 4)
    l_commit = commit[0, 0] / jnp.float32(_B * _R * _C)
    l_codebook = jnp.zeros((), jnp.float32)
    return (vecs_hat, z_out, l_commit, l_codebook, e_out)


# speedup vs baseline: 1.0503x; 1.0503x over previous
import jax
import jax.numpy as jnp
from jax import lax
from jax.experimental import pallas as pl
from jax.experimental.pallas import tpu as pltpu

_B, _H, _R, _C, _K, _S = 2, 8, 16, 128, 64, 512
_N = _B * _R * _C


def _vq_body(vecs_ref, csum_ref, ccnt_ref, vq_ref, z_ref, e_ref, commit_ref):
    h = pl.program_id(0)

    c = csum_ref[0] / jnp.maximum(ccnt_ref[0], 0.01)
    c2 = -2.0 * c
    cn = jnp.sum(c * c, axis=1, keepdims=True)
    v = jnp.concatenate(
        [vecs_ref[b, 0, r] for b in range(_B) for r in range(_R)],
        axis=1)
    dot2 = lax.dot_general(c2, v, (((1,), (0,)), ((), ())),
                           preferred_element_type=jnp.float32)
    vnorm = jnp.sum(v * v, axis=0, keepdims=True)
    d2 = (vnorm + dot2) + cn
    mind = jnp.min(d2, axis=0, keepdims=True)
    iota_col = lax.broadcasted_iota(jnp.int32, (_S, 1), 0).astype(jnp.float32)
    zf = jnp.min(jnp.where(d2 == mind, iota_col, jnp.float32(_S)),
                 axis=0, keepdims=True)
    onehot = (iota_col == zf).astype(jnp.float32)
    cz = lax.dot_general(c, onehot, (((0,), (0,)), ((), ())),
                         preferred_element_type=jnp.float32)
    for b in range(_B):
        for r in range(_R):
            n0 = (b * _R + r) * _C
            vq_ref[b, 0, r] = cz[:, n0:n0 + _C]
    z_ref[...] = zf.astype(jnp.int32).reshape(_B, 1, _R, _C)
    e = jnp.maximum(mind, 0.0)
    e_ref[...] = e.reshape(_B, 1, _R, _C)
    prev = jnp.where(h == 0, 0.0, commit_ref[0, 0])
    commit_ref[0, 0] = prev + jnp.sum(e)


def kernel(vecs, c_sum, c_count):
    vt = jnp.swapaxes(vecs, 3, 4)
    ccnt = c_count.reshape(_H, _S, 1)

    def im_v(h):
        return (0, h, 0, 0, 0)

    def im_cb(h):
        return (h, 0, 0)

    def im_ze(h):
        return (0, h, 0, 0)

    vq, z_out, e_out, commit = pl.pallas_call(
        _vq_body,
        grid=(_H,),
        in_specs=[
            pl.BlockSpec((_B, 1, _R, _K, _C), im_v),
            pl.BlockSpec((1, _S, _K), im_cb),
            pl.BlockSpec((1, _S, 1), im_cb),
        ],
        out_specs=[
            pl.BlockSpec((_B, 1, _R, _K, _C), im_v),
            pl.BlockSpec((_B, 1, _R, _C), im_ze),
            pl.BlockSpec((_B, 1, _R, _C), im_ze),
            pl.BlockSpec((1, 1), lambda h: (0, 0), memory_space=pltpu.SMEM),
        ],
        out_shape=[
            jax.ShapeDtypeStruct((_B, _H, _R, _K, _C), jnp.float32),
            jax.ShapeDtypeStruct((_B, _H, _R, _C), jnp.int32),
            jax.ShapeDtypeStruct((_B, _H, _R, _C), jnp.float32),
            jax.ShapeDtypeStruct((1, 1), jnp.float32),
        ],
    )(vt, c_sum, ccnt)

    vecs_hat = jnp.swapaxes(vq, 3, 4)
    l_commit = commit[0, 0] / jnp.float32(_B * _R * _C)
    l_codebook = jnp.zeros((), jnp.float32)
    return (vecs_hat, z_out, l_commit, l_codebook, e_out)
